# baseline (speedup 1.0000x reference)
import dataclasses
import functools

import jax
import jax.numpy as jnp
from jax import lax
from jax.experimental import pallas as pl
from jax.experimental.pallas import tpu as pltpu
from jax.experimental.pallas import tpu_sc as plsc

N_NODES = 10000
N_EDGES = 320000
D_IN = 128
D_HID = 128
D_OUT = 64

NC = 2
NS = 16
N_PAD = 10240
EDGE_BATCH = 128
BATCHES_PER_TILE = 161
EDGES_PER_TILE = BATCHES_PER_TILE * EDGE_BATCH
E_PAD = EDGES_PER_TILE * NS


def _make_spmm(dh):
    grp = dh // 16
    rows_per_tile = N_PAD // NS
    mesh = plsc.VectorSubcoreMesh(core_axis_name="c", subcore_axis_name="s")
    cp = pltpu.CompilerParams()
    if "needs_layout_passes" in pltpu.CompilerParams.__dataclass_fields__:
        cp = dataclasses.replace(cp, needs_layout_passes=False)
    if "use_tc_tiling_on_sc" in pltpu.CompilerParams.__dataclass_fields__:
        cp = dataclasses.replace(cp, use_tc_tiling_on_sc=False)

    NB = BATCHES_PER_TILE

    @functools.partial(
        pl.kernel,
        out_type=jax.ShapeDtypeStruct((NC, N_PAD, dh), jnp.float32),
        mesh=mesh,
        compiler_params=cp,
        scratch_types=[
            pltpu.VMEM((6, 3, EDGE_BATCH), jnp.int32),
            pltpu.VMEM((3, EDGE_BATCH, dh), jnp.float32),
            pltpu.VMEM_SHARED((N_PAD, dh), jnp.float32),
            pltpu.VMEM_SHARED((N_PAD, dh), jnp.float32),
        ] + [pltpu.SemaphoreType.DMA] * 12,
    )
    def spmm(xh_hbm, ep_hbm, out_hbm, ibuf, rows, acc, xbuf,
             si0, si1, si2, si3, si4, si5, sg0, sg1, sg2, ss0, ss1, ss2):
        c = lax.axis_index("c")
        s = lax.axis_index("s")
        sems_i = [si0, si1, si2, si3, si4, si5]
        sems_g = [sg0, sg1, sg2]
        sems_s = [ss0, ss1, ss2]
        zero = jnp.zeros((16,), jnp.float32)
        base_row = s * rows_per_tile
        xc = xh_hbm.at[c]

        def idx_issue(g, s6):
            pltpu.async_copy(ep_hbm.at[s, g], ibuf.at[s6], sems_i[s6])

        def idx_wait(g, s6):
            pltpu.make_async_copy(ep_hbm.at[s, g], ibuf.at[s6], sems_i[s6]).wait()

        def gather(g, s3, s6):
            pltpu.async_copy(xbuf.at[ibuf.at[s6, 0]], rows.at[s3], sems_g[s3])

        def wait_gather(g, s3, s6):
            pltpu.make_async_copy(xbuf.at[ibuf.at[s6, 0]], rows.at[s3],
                                  sems_g[s3]).wait()

        def scatter(g, s3, s6):
            pltpu.async_copy(rows.at[s3], acc.at[ibuf.at[s6, 1]], sems_s[s3],
                             add=True)

        def wait_scatter(g, s3, s6):
            pltpu.make_async_copy(rows.at[s3], acc.at[ibuf.at[s6, 1]],
                                  sems_s[s3]).wait()

        splat_idx = [jnp.full((16,), i, jnp.int32) for i in range(16)]

        def scale(g, s3, s6):
            rs = rows.at[s3]

            @plsc.parallel_loop(0, EDGE_BATCH // 16, unroll=4)
            def _scale(j):
                w16 = plsc.bitcast(ibuf[s6, 2, pl.ds(j * 16, 16)], jnp.float32)
                for i in range(16):
                    wi = w16.at[splat_idx[i]].get(mode="promise_in_bounds")
                    e = j * 16 + i
                    for f in range(grp):
                        rs[e, pl.ds(f * 16, 16)] = rs[e, pl.ds(f * 16, 16)] * wi

        def step(g, s3, s6, wait_prev, issue_idx, issue_gather):
            if wait_prev:
                wait_scatter(g - 1, (s3 + 2) % 3, (s6 + 5) % 6)
            if issue_idx:
                idx_issue(g + 4, (s6 + 4) % 6)
            if issue_gather:
                idx_wait(g + 2, (s6 + 2) % 6)
                gather(g + 2, (s3 + 2) % 3, (s6 + 2) % 6)
            wait_gather(g, s3, s6)
            scale(g, s3, s6)
            scatter(g, s3, s6)

        for g0 in range(4):
            idx_issue(g0, g0)
        pltpu.sync_copy(xc.at[pl.ds(base_row, rows_per_tile)],
                        xbuf.at[pl.ds(base_row, rows_per_tile)])

        r0 = rows.at[0]

        @pl.loop(0, EDGE_BATCH)
        def _zero_rows(r):
            for f in range(grp):
                r0[r, pl.ds(f * 16, 16)] = zero

        for k in range(rows_per_tile // EDGE_BATCH):
            pltpu.sync_copy(r0, acc.at[pl.ds(base_row + k * EDGE_BATCH, EDGE_BATCH)])
        plsc.subcore_barrier()

        idx_wait(0, 0)
        gather(0, 0, 0)
        idx_wait(1, 1)
        gather(1, 1, 1)

        step(0, 0, 0, False, True, True)

        @pl.loop(0, (NB - 5) // 6)
        def _main(it):
            base = 6 * it + 1
            for k in range(6):
                g = base + k
                step(g, (1 + k) % 3, (1 + k) % 6, True, True, True)

        step(NB - 4, (NB - 4) % 3, (NB - 4) % 6, True, False, True)
        step(NB - 3, (NB - 3) % 3, (NB - 3) % 6, True, False, True)
        step(NB - 2, (NB - 2) % 3, (NB - 2) % 6, True, False, False)
        step(NB - 1, (NB - 1) % 3, (NB - 1) % 6, True, False, False)
        wait_scatter(NB - 1, (NB - 1) % 3, (NB - 1) % 6)

        plsc.subcore_barrier()
        pltpu.sync_copy(acc.at[pl.ds(base_row, rows_per_tile)],
                        out_hbm.at[c, pl.ds(base_row, rows_per_tile)])

    return spmm


_spmm_hid = _make_spmm(D_HID // 2)
_spmm_out = _make_spmm(D_OUT // 2)


def _dense_body(p_ref, w0_ref, b0_ref, w1_ref, b1_ref, y_ref):
    p = jnp.concatenate([p_ref[0], p_ref[1]], axis=1)
    h = jnp.maximum(
        jnp.dot(p, w0_ref[...], preferred_element_type=jnp.float32) + b0_ref[...], 0.0)
    y = jnp.dot(h, w1_ref[...], preferred_element_type=jnp.float32) + b1_ref[...]
    y_ref[0] = y[:, :D_OUT // 2]
    y_ref[1] = y[:, D_OUT // 2:]


def _dense(p, w0, b0, w1, b1):
    rows = 2048
    grid = (N_PAD // rows,)
    return pl.pallas_call(
        _dense_body,
        grid=grid,
        in_specs=[
            pl.BlockSpec((NC, rows, D_IN // 2), lambda i: (0, i, 0)),
            pl.BlockSpec((D_IN, D_HID), lambda i: (0, 0)),
            pl.BlockSpec((1, D_HID), lambda i: (0, 0)),
            pl.BlockSpec((D_HID, D_OUT), lambda i: (0, 0)),
            pl.BlockSpec((1, D_OUT), lambda i: (0, 0)),
        ],
        out_specs=pl.BlockSpec((NC, rows, D_OUT // 2), lambda i: (0, i, 0)),
        out_shape=jax.ShapeDtypeStruct((NC, N_PAD, D_OUT // 2), jnp.float32),
    )(p, w0, b0, w1, b1)


def _logsoftmax_body(q_ref, o_ref):
    z = jnp.concatenate([q_ref[0], q_ref[1]], axis=1)
    z = z - jnp.max(z, axis=1, keepdims=True)
    o_ref[...] = z - jnp.log(jnp.sum(jnp.exp(z), axis=1, keepdims=True))


def _logsoftmax(q):
    rows = 2000
    grid = (N_NODES // rows,)
    return pl.pallas_call(
        _logsoftmax_body,
        grid=grid,
        in_specs=[pl.BlockSpec((NC, rows, D_OUT // 2), lambda i: (0, i, 0))],
        out_specs=pl.BlockSpec((rows, D_OUT), lambda i: (i, 0)),
        out_shape=jax.ShapeDtypeStruct((N_NODES, D_OUT), jnp.float32),
    )(q)


def kernel(x, edge_index, edge_weight, W0, b0, W1, b1):
    src = edge_index[0].astype(jnp.int32)
    dst = edge_index[1].astype(jnp.int32)
    w_i = lax.bitcast_convert_type(edge_weight.astype(jnp.float32), jnp.int32)
    pad = E_PAD - N_EDGES
    src = jnp.concatenate([src, jnp.zeros((pad,), jnp.int32)])
    dst = jnp.concatenate([dst, jnp.zeros((pad,), jnp.int32)])
    w_i = jnp.concatenate([w_i, jnp.zeros((pad,), jnp.int32)])
    ep = (jnp.stack([src, dst, w_i], axis=0)
          .reshape(3, NS, BATCHES_PER_TILE, EDGE_BATCH)
          .transpose(1, 2, 0, 3))

    xh = x.reshape(N_NODES, NC, D_IN // 2).transpose(1, 0, 2)
    xh = jnp.concatenate(
        [xh, jnp.zeros((NC, N_PAD - N_NODES, D_IN // 2), jnp.float32)], axis=1)
    p = _spmm_hid(xh, ep)
    y = _dense(p, W0, b0.reshape(1, D_HID), W1, b1.reshape(1, D_OUT))
    q = _spmm_out(y, ep)
    return _logsoftmax(q)


# speedup vs baseline: 1.1150x; 1.1150x over previous
import dataclasses
import functools

import jax
import jax.numpy as jnp
from jax import lax
from jax.experimental import pallas as pl
from jax.experimental.pallas import tpu as pltpu
from jax.experimental.pallas import tpu_sc as plsc

N_NODES = 10000
N_EDGES = 320000
D_IN = 128
D_HID = 128
D_OUT = 64

NC = 2
NS = 16
N_PAD = 10240
EDGE_BATCH = 128
BATCHES_PER_TILE = 161
EDGES_PER_TILE = BATCHES_PER_TILE * EDGE_BATCH
E_PAD = EDGES_PER_TILE * NS


def _make_spmm(dh):
    grp = dh // 16
    rows_per_tile = N_PAD // NS
    mesh = plsc.VectorSubcoreMesh(core_axis_name="c", subcore_axis_name="s")
    cp = pltpu.CompilerParams()
    if "needs_layout_passes" in pltpu.CompilerParams.__dataclass_fields__:
        cp = dataclasses.replace(cp, needs_layout_passes=False)
    if "use_tc_tiling_on_sc" in pltpu.CompilerParams.__dataclass_fields__:
        cp = dataclasses.replace(cp, use_tc_tiling_on_sc=False)

    NB = BATCHES_PER_TILE

    @functools.partial(
        pl.kernel,
        out_type=jax.ShapeDtypeStruct((NC, N_PAD, dh), jnp.float32),
        mesh=mesh,
        compiler_params=cp,
        scratch_types=[
            pltpu.VMEM((6, 3, EDGE_BATCH), jnp.int32),
            pltpu.VMEM((3, EDGE_BATCH, dh), jnp.float32),
            pltpu.VMEM_SHARED((N_PAD, dh), jnp.float32),
            pltpu.VMEM_SHARED((N_PAD, dh), jnp.float32),
        ] + [pltpu.SemaphoreType.DMA] * 12,
    )
    def spmm(xh_hbm, ep_hbm, out_hbm, ibuf, rows, acc, xbuf,
             si0, si1, si2, si3, si4, si5, sg0, sg1, sg2, ss0, ss1, ss2):
        c = lax.axis_index("c")
        s = lax.axis_index("s")
        sems_i = [si0, si1, si2, si3, si4, si5]
        sems_g = [sg0, sg1, sg2]
        sems_s = [ss0, ss1, ss2]
        zero = jnp.zeros((16,), jnp.float32)
        base_row = s * rows_per_tile
        xc = xh_hbm.at[c]

        def idx_issue(g, s6):
            pltpu.async_copy(ep_hbm.at[s, g], ibuf.at[s6], sems_i[s6])

        def idx_wait(g, s6):
            pltpu.make_async_copy(ep_hbm.at[s, g], ibuf.at[s6], sems_i[s6]).wait()

        def gather(g, s3, s6):
            pltpu.async_copy(xbuf.at[ibuf.at[s6, 0]], rows.at[s3], sems_g[s3])

        def wait_gather(g, s3, s6):
            pltpu.make_async_copy(xbuf.at[ibuf.at[s6, 0]], rows.at[s3],
                                  sems_g[s3]).wait()

        def scatter(g, s3, s6):
            pltpu.async_copy(rows.at[s3], acc.at[ibuf.at[s6, 1]], sems_s[s3],
                             add=True)

        def wait_scatter(g, s3, s6):
            pltpu.make_async_copy(rows.at[s3], acc.at[ibuf.at[s6, 1]],
                                  sems_s[s3]).wait()

        splat_idx = [jnp.full((16,), i, jnp.int32) for i in range(16)]

        def scale(g, s3, s6):
            rs = rows.at[s3]

            @plsc.parallel_loop(0, EDGE_BATCH // 16, unroll=2)
            def _scale(j):
                w16 = plsc.bitcast(ibuf[s6, 2, pl.ds(j * 16, 16)], jnp.float32)
                for i in range(16):
                    wi = w16.at[splat_idx[i]].get(mode="promise_in_bounds")
                    e = j * 16 + i
                    for f in range(grp):
                        rs[e, pl.ds(f * 16, 16)] = rs[e, pl.ds(f * 16, 16)] * wi

        def step(g, s3, s6, wait_prev, issue_idx, issue_gather):
            if issue_idx:
                idx_issue(g + 4, (s6 + 4) % 6)
            wait_gather(g, s3, s6)
            scale(g, s3, s6)
            if wait_prev:
                wait_scatter(g - 1, (s3 + 2) % 3, (s6 + 5) % 6)
            if issue_gather:
                idx_wait(g + 2, (s6 + 2) % 6)
                gather(g + 2, (s3 + 2) % 3, (s6 + 2) % 6)
            scatter(g, s3, s6)

        for g0 in range(4):
            idx_issue(g0, g0)
        pltpu.sync_copy(xc.at[pl.ds(base_row, rows_per_tile)],
                        xbuf.at[pl.ds(base_row, rows_per_tile)])

        r0 = rows.at[0]

        @pl.loop(0, EDGE_BATCH)
        def _zero_rows(r):
            for f in range(grp):
                r0[r, pl.ds(f * 16, 16)] = zero

        for k in range(rows_per_tile // EDGE_BATCH):
            pltpu.sync_copy(r0, acc.at[pl.ds(base_row + k * EDGE_BATCH, EDGE_BATCH)])
        plsc.subcore_barrier()

        idx_wait(0, 0)
        gather(0, 0, 0)
        idx_wait(1, 1)
        gather(1, 1, 1)

        step(0, 0, 0, False, True, True)

        @pl.loop(0, (NB - 5) // 6)
        def _main(it):
            base = 6 * it + 1
            for k in range(6):
                g = base + k
                step(g, (1 + k) % 3, (1 + k) % 6, True, True, True)

        step(NB - 4, (NB - 4) % 3, (NB - 4) % 6, True, False, True)
        step(NB - 3, (NB - 3) % 3, (NB - 3) % 6, True, False, True)
        step(NB - 2, (NB - 2) % 3, (NB - 2) % 6, True, False, False)
        step(NB - 1, (NB - 1) % 3, (NB - 1) % 6, True, False, False)
        wait_scatter(NB - 1, (NB - 1) % 3, (NB - 1) % 6)

        plsc.subcore_barrier()
        pltpu.sync_copy(acc.at[pl.ds(base_row, rows_per_tile)],
                        out_hbm.at[c, pl.ds(base_row, rows_per_tile)])

    return spmm


_spmm_hid = _make_spmm(D_HID // 2)
_spmm_out = _make_spmm(D_OUT // 2)


def _dense_body(p_ref, w0_ref, b0_ref, w1_ref, b1_ref, y_ref):
    p = jnp.concatenate([p_ref[0], p_ref[1]], axis=1)
    h = jnp.maximum(
        jnp.dot(p, w0_ref[...], preferred_element_type=jnp.float32) + b0_ref[...], 0.0)
    y = jnp.dot(h, w1_ref[...], preferred_element_type=jnp.float32) + b1_ref[...]
    y_ref[0] = y[:, :D_OUT // 2]
    y_ref[1] = y[:, D_OUT // 2:]


def _dense(p, w0, b0, w1, b1):
    rows = 2048
    grid = (N_PAD // rows,)
    return pl.pallas_call(
        _dense_body,
        grid=grid,
        in_specs=[
            pl.BlockSpec((NC, rows, D_IN // 2), lambda i: (0, i, 0)),
            pl.BlockSpec((D_IN, D_HID), lambda i: (0, 0)),
            pl.BlockSpec((1, D_HID), lambda i: (0, 0)),
            pl.BlockSpec((D_HID, D_OUT), lambda i: (0, 0)),
            pl.BlockSpec((1, D_OUT), lambda i: (0, 0)),
        ],
        out_specs=pl.BlockSpec((NC, rows, D_OUT // 2), lambda i: (0, i, 0)),
        out_shape=jax.ShapeDtypeStruct((NC, N_PAD, D_OUT // 2), jnp.float32),
    )(p, w0, b0, w1, b1)


def _logsoftmax_body(q_ref, o_ref):
    z = jnp.concatenate([q_ref[0], q_ref[1]], axis=1)
    z = z - jnp.max(z, axis=1, keepdims=True)
    o_ref[...] = z - jnp.log(jnp.sum(jnp.exp(z), axis=1, keepdims=True))


def _logsoftmax(q):
    rows = 2000
    grid = (N_NODES // rows,)
    return pl.pallas_call(
        _logsoftmax_body,
        grid=grid,
        in_specs=[pl.BlockSpec((NC, rows, D_OUT // 2), lambda i: (0, i, 0))],
        out_specs=pl.BlockSpec((rows, D_OUT), lambda i: (i, 0)),
        out_shape=jax.ShapeDtypeStruct((N_NODES, D_OUT), jnp.float32),
    )(q)


def kernel(x, edge_index, edge_weight, W0, b0, W1, b1):
    src = edge_index[0].astype(jnp.int32)
    dst = edge_index[1].astype(jnp.int32)
    w_i = lax.bitcast_convert_type(edge_weight.astype(jnp.float32), jnp.int32)
    pad = E_PAD - N_EDGES
    src = jnp.concatenate([src, jnp.zeros((pad,), jnp.int32)])
    dst = jnp.concatenate([dst, jnp.zeros((pad,), jnp.int32)])
    w_i = jnp.concatenate([w_i, jnp.zeros((pad,), jnp.int32)])
    ep = (jnp.stack([src, dst, w_i], axis=0)
          .reshape(3, NS, BATCHES_PER_TILE, EDGE_BATCH)
          .transpose(1, 2, 0, 3))

    xh = x.reshape(N_NODES, NC, D_IN // 2).transpose(1, 0, 2)
    xh = jnp.concatenate(
        [xh, jnp.zeros((NC, N_PAD - N_NODES, D_IN // 2), jnp.float32)], axis=1)
    p = _spmm_hid(xh, ep)
    y = _dense(p, W0, b0.reshape(1, D_HID), W1, b1.reshape(1, D_OUT))
    q = _spmm_out(y, ep)
    return _logsoftmax(q)


# speedup vs baseline: 1.2475x; 1.1188x over previous
import dataclasses
import functools

import jax
import jax.numpy as jnp
from jax import lax
from jax.experimental import pallas as pl
from jax.experimental.pallas import tpu as pltpu
from jax.experimental.pallas import tpu_sc as plsc

N_NODES = 10000
N_EDGES = 320000
D_IN = 128
D_HID = 128
D_OUT = 64

NC = 2
NS = 16
N_PAD = 10240
EDGE_BATCH = 128
BATCHES_PER_TILE = 158
EDGES_PER_TILE = BATCHES_PER_TILE * EDGE_BATCH
E_PAD = EDGES_PER_TILE * NS


def _make_spmm(dh):
    grp = dh // 16
    rows_per_tile = N_PAD // NS
    mesh = plsc.VectorSubcoreMesh(core_axis_name="c", subcore_axis_name="s")
    cp = pltpu.CompilerParams()
    if "needs_layout_passes" in pltpu.CompilerParams.__dataclass_fields__:
        cp = dataclasses.replace(cp, needs_layout_passes=False)
    if "use_tc_tiling_on_sc" in pltpu.CompilerParams.__dataclass_fields__:
        cp = dataclasses.replace(cp, use_tc_tiling_on_sc=False)

    NB = BATCHES_PER_TILE

    @functools.partial(
        pl.kernel,
        out_type=jax.ShapeDtypeStruct((NC, N_PAD, dh), jnp.float32),
        mesh=mesh,
        compiler_params=cp,
        scratch_types=[
            pltpu.VMEM((8, 3, EDGE_BATCH), jnp.int32),
            pltpu.VMEM((4, EDGE_BATCH, dh), jnp.float32),
            pltpu.VMEM_SHARED((N_PAD, dh), jnp.float32),
            pltpu.VMEM_SHARED((N_PAD, dh), jnp.float32),
        ] + [pltpu.SemaphoreType.DMA] * 16,
    )
    def spmm(xh_hbm, ep_hbm, out_hbm, ibuf, rows, acc, xbuf,
             si0, si1, si2, si3, si4, si5, si6, si7,
             sg0, sg1, sg2, sg3, ss0, ss1, ss2, ss3):
        c = lax.axis_index("c")
        s = lax.axis_index("s")
        sems_i = [si0, si1, si2, si3, si4, si5, si6, si7]
        sems_g = [sg0, sg1, sg2, sg3]
        sems_s = [ss0, ss1, ss2, ss3]
        zero = jnp.zeros((16,), jnp.float32)
        base_row = s * rows_per_tile
        xc = xh_hbm.at[c]

        def idx_issue(g, s6):
            pltpu.async_copy(ep_hbm.at[s, g], ibuf.at[s6], sems_i[s6])

        def idx_wait(g, s6):
            pltpu.make_async_copy(ep_hbm.at[s, g], ibuf.at[s6], sems_i[s6]).wait()

        def gather(g, s3, s6):
            pltpu.async_copy(xbuf.at[ibuf.at[s6, 0]], rows.at[s3], sems_g[s3])

        def wait_gather(g, s3, s6):
            pltpu.make_async_copy(xbuf.at[ibuf.at[s6, 0]], rows.at[s3],
                                  sems_g[s3]).wait()

        def scatter(g, s3, s6):
            pltpu.async_copy(rows.at[s3], acc.at[ibuf.at[s6, 1]], sems_s[s3],
                             add=True)

        def wait_scatter(g, s3, s6):
            pltpu.make_async_copy(rows.at[s3], acc.at[ibuf.at[s6, 1]],
                                  sems_s[s3]).wait()

        splat_idx = [jnp.full((16,), i, jnp.int32) for i in range(16)]

        def scale(g, s3, s6):
            rs = rows.at[s3]

            @plsc.parallel_loop(0, EDGE_BATCH // 16, unroll=2)
            def _scale(j):
                w16 = plsc.bitcast(ibuf[s6, 2, pl.ds(j * 16, 16)], jnp.float32)
                for i in range(16):
                    wi = w16.at[splat_idx[i]].get(mode="promise_in_bounds")
                    e = j * 16 + i
                    for f in range(grp):
                        rs[e, pl.ds(f * 16, 16)] = rs[e, pl.ds(f * 16, 16)] * wi

        def step(g, s4, s8, wait_prev, issue_idx, issue_gather):
            if issue_idx:
                idx_issue(g + 4, (s8 + 4) % 8)
            wait_gather(g, s4, s8)
            scale(g, s4, s8)
            if wait_prev:
                wait_scatter(g - 2, (s4 + 2) % 4, (s8 + 6) % 8)
            if issue_gather:
                idx_wait(g + 2, (s8 + 2) % 8)
                gather(g + 2, (s4 + 2) % 4, (s8 + 2) % 8)
            scatter(g, s4, s8)

        for g0 in range(4):
            idx_issue(g0, g0)
        pltpu.sync_copy(xc.at[pl.ds(base_row, rows_per_tile)],
                        xbuf.at[pl.ds(base_row, rows_per_tile)])

        r0 = rows.at[0]

        @pl.loop(0, EDGE_BATCH)
        def _zero_rows(r):
            for f in range(grp):
                r0[r, pl.ds(f * 16, 16)] = zero

        for k in range(rows_per_tile // EDGE_BATCH):
            pltpu.sync_copy(r0, acc.at[pl.ds(base_row + k * EDGE_BATCH, EDGE_BATCH)])
        plsc.subcore_barrier()

        idx_wait(0, 0)
        gather(0, 0, 0)
        idx_wait(1, 1)
        gather(1, 1, 1)

        step(0, 0, 0, False, True, True)
        step(1, 1, 1, False, True, True)

        @pl.loop(0, (NB - 6) // 8)
        def _main(it):
            base = 8 * it + 2
            for k in range(8):
                g = base + k
                step(g, (2 + k) % 4, (2 + k) % 8, True, True, True)

        step(NB - 4, (NB - 4) % 4, (NB - 4) % 8, True, False, True)
        step(NB - 3, (NB - 3) % 4, (NB - 3) % 8, True, False, True)
        step(NB - 2, (NB - 2) % 4, (NB - 2) % 8, True, False, False)
        step(NB - 1, (NB - 1) % 4, (NB - 1) % 8, True, False, False)
        wait_scatter(NB - 2, (NB - 2) % 4, (NB - 2) % 8)
        wait_scatter(NB - 1, (NB - 1) % 4, (NB - 1) % 8)

        plsc.subcore_barrier()
        pltpu.sync_copy(acc.at[pl.ds(base_row, rows_per_tile)],
                        out_hbm.at[c, pl.ds(base_row, rows_per_tile)])

    return spmm


_spmm_hid = _make_spmm(D_HID // 2)
_spmm_out = _make_spmm(D_OUT // 2)


def _dense_body(p_ref, w0_ref, b0_ref, w1_ref, b1_ref, y_ref):
    p = jnp.concatenate([p_ref[0], p_ref[1]], axis=1)
    h = jnp.maximum(
        jnp.dot(p, w0_ref[...], preferred_element_type=jnp.float32) + b0_ref[...], 0.0)
    y = jnp.dot(h, w1_ref[...], preferred_element_type=jnp.float32) + b1_ref[...]
    y_ref[0] = y[:, :D_OUT // 2]
    y_ref[1] = y[:, D_OUT // 2:]


def _dense(p, w0, b0, w1, b1):
    rows = 2048
    grid = (N_PAD // rows,)
    return pl.pallas_call(
        _dense_body,
        grid=grid,
        in_specs=[
            pl.BlockSpec((NC, rows, D_IN // 2), lambda i: (0, i, 0)),
            pl.BlockSpec((D_IN, D_HID), lambda i: (0, 0)),
            pl.BlockSpec((1, D_HID), lambda i: (0, 0)),
            pl.BlockSpec((D_HID, D_OUT), lambda i: (0, 0)),
            pl.BlockSpec((1, D_OUT), lambda i: (0, 0)),
        ],
        out_specs=pl.BlockSpec((NC, rows, D_OUT // 2), lambda i: (0, i, 0)),
        out_shape=jax.ShapeDtypeStruct((NC, N_PAD, D_OUT // 2), jnp.float32),
    )(p, w0, b0, w1, b1)


def _logsoftmax_body(q_ref, o_ref):
    z = jnp.concatenate([q_ref[0], q_ref[1]], axis=1)
    z = z - jnp.max(z, axis=1, keepdims=True)
    o_ref[...] = z - jnp.log(jnp.sum(jnp.exp(z), axis=1, keepdims=True))


def _logsoftmax(q):
    rows = 2000
    grid = (N_NODES // rows,)
    return pl.pallas_call(
        _logsoftmax_body,
        grid=grid,
        in_specs=[pl.BlockSpec((NC, rows, D_OUT // 2), lambda i: (0, i, 0))],
        out_specs=pl.BlockSpec((rows, D_OUT), lambda i: (i, 0)),
        out_shape=jax.ShapeDtypeStruct((N_NODES, D_OUT), jnp.float32),
    )(q)


def kernel(x, edge_index, edge_weight, W0, b0, W1, b1):
    src = edge_index[0].astype(jnp.int32)
    dst = edge_index[1].astype(jnp.int32)
    w_i = lax.bitcast_convert_type(edge_weight.astype(jnp.float32), jnp.int32)
    pad = E_PAD - N_EDGES
    src = jnp.concatenate([src, jnp.zeros((pad,), jnp.int32)])
    dst = jnp.concatenate([dst, jnp.zeros((pad,), jnp.int32)])
    w_i = jnp.concatenate([w_i, jnp.zeros((pad,), jnp.int32)])
    ep = (jnp.stack([src, dst, w_i], axis=0)
          .reshape(3, NS, BATCHES_PER_TILE, EDGE_BATCH)
          .transpose(1, 2, 0, 3))

    xh = x.reshape(N_NODES, NC, D_IN // 2).transpose(1, 0, 2)
    xh = jnp.concatenate(
        [xh, jnp.zeros((NC, N_PAD - N_NODES, D_IN // 2), jnp.float32)], axis=1)
    p = _spmm_hid(xh, ep)
    y = _dense(p, W0, b0.reshape(1, D_HID), W1, b1.reshape(1, D_OUT))
    q = _spmm_out(y, ep)
    return _logsoftmax(q)


# speedup vs baseline: 1.2494x; 1.0016x over previous
import dataclasses
import functools

import jax
import jax.numpy as jnp
from jax import lax
from jax.experimental import pallas as pl
from jax.experimental.pallas import tpu as pltpu
from jax.experimental.pallas import tpu_sc as plsc

N_NODES = 10000
N_EDGES = 320000
D_IN = 128
D_HID = 128
D_OUT = 64

NC = 2
NS = 16
N_PAD = 10240
EDGE_BATCH = 128
BATCHES_PER_TILE = 157
EDGES_PER_TILE = BATCHES_PER_TILE * EDGE_BATCH
E_PAD = EDGES_PER_TILE * NS


def _make_spmm(dh):
    grp = dh // 16
    rows_per_tile = N_PAD // NS
    mesh = plsc.VectorSubcoreMesh(core_axis_name="c", subcore_axis_name="s")
    cp = pltpu.CompilerParams()
    if "needs_layout_passes" in pltpu.CompilerParams.__dataclass_fields__:
        cp = dataclasses.replace(cp, needs_layout_passes=False)
    if "use_tc_tiling_on_sc" in pltpu.CompilerParams.__dataclass_fields__:
        cp = dataclasses.replace(cp, use_tc_tiling_on_sc=False)

    NB = BATCHES_PER_TILE

    @functools.partial(
        pl.kernel,
        out_type=jax.ShapeDtypeStruct((NC, N_PAD, dh), jnp.float32),
        mesh=mesh,
        compiler_params=cp,
        scratch_types=[
            pltpu.VMEM((10, 3, EDGE_BATCH), jnp.int32),
            pltpu.VMEM((5, EDGE_BATCH, dh), jnp.float32),
            pltpu.VMEM_SHARED((N_PAD, dh), jnp.float32),
            pltpu.VMEM_SHARED((N_PAD, dh), jnp.float32),
        ] + [pltpu.SemaphoreType.DMA] * 20,
    )
    def spmm(xh_hbm, ep_hbm, out_hbm, ibuf, rows, acc, xbuf,
             si0, si1, si2, si3, si4, si5, si6, si7, si8, si9,
             sg0, sg1, sg2, sg3, sg4, ss0, ss1, ss2, ss3, ss4):
        c = lax.axis_index("c")
        s = lax.axis_index("s")
        sems_i = [si0, si1, si2, si3, si4, si5, si6, si7, si8, si9]
        sems_g = [sg0, sg1, sg2, sg3, sg4]
        sems_s = [ss0, ss1, ss2, ss3, ss4]
        zero = jnp.zeros((16,), jnp.float32)
        base_row = s * rows_per_tile
        xc = xh_hbm.at[c]

        def idx_issue(g, s6):
            pltpu.async_copy(ep_hbm.at[s, g], ibuf.at[s6], sems_i[s6])

        def idx_wait(g, s6):
            pltpu.make_async_copy(ep_hbm.at[s, g], ibuf.at[s6], sems_i[s6]).wait()

        def gather(g, s3, s6):
            pltpu.async_copy(xbuf.at[ibuf.at[s6, 0]], rows.at[s3], sems_g[s3])

        def wait_gather(g, s3, s6):
            pltpu.make_async_copy(xbuf.at[ibuf.at[s6, 0]], rows.at[s3],
                                  sems_g[s3]).wait()

        def scatter(g, s3, s6):
            pltpu.async_copy(rows.at[s3], acc.at[ibuf.at[s6, 1]], sems_s[s3],
                             add=True)

        def wait_scatter(g, s3, s6):
            pltpu.make_async_copy(rows.at[s3], acc.at[ibuf.at[s6, 1]],
                                  sems_s[s3]).wait()

        splat_idx = [jnp.full((16,), i, jnp.int32) for i in range(16)]

        def scale(g, s3, s6):
            rs = rows.at[s3]

            @plsc.parallel_loop(0, EDGE_BATCH // 16, unroll=2)
            def _scale(j):
                w16 = plsc.bitcast(ibuf[s6, 2, pl.ds(j * 16, 16)], jnp.float32)
                for i in range(16):
                    wi = w16.at[splat_idx[i]].get(mode="promise_in_bounds")
                    e = j * 16 + i
                    for f in range(grp):
                        rs[e, pl.ds(f * 16, 16)] = rs[e, pl.ds(f * 16, 16)] * wi

        def step(g, s5, s10, wait_prev, issue_idx, issue_gather):
            if issue_idx:
                idx_issue(g + 4, (s10 + 4) % 10)
            wait_gather(g, s5, s10)
            scale(g, s5, s10)
            if wait_prev:
                wait_scatter(g - 3, (s5 + 2) % 5, (s10 + 7) % 10)
            if issue_gather:
                idx_wait(g + 2, (s10 + 2) % 10)
                gather(g + 2, (s5 + 2) % 5, (s10 + 2) % 10)
            scatter(g, s5, s10)

        for g0 in range(4):
            idx_issue(g0, g0)
        pltpu.sync_copy(xc.at[pl.ds(base_row, rows_per_tile)],
                        xbuf.at[pl.ds(base_row, rows_per_tile)])

        r0 = rows.at[0]

        @pl.loop(0, EDGE_BATCH)
        def _zero_rows(r):
            for f in range(grp):
                r0[r, pl.ds(f * 16, 16)] = zero

        for k in range(rows_per_tile // EDGE_BATCH):
            pltpu.sync_copy(r0, acc.at[pl.ds(base_row + k * EDGE_BATCH, EDGE_BATCH)])
        plsc.subcore_barrier()

        idx_wait(0, 0)
        gather(0, 0, 0)
        idx_wait(1, 1)
        gather(1, 1, 1)

        step(0, 0, 0, False, True, True)
        step(1, 1, 1, False, True, True)
        step(2, 2, 2, False, True, True)

        @pl.loop(0, (NB - 7) // 10)
        def _main(it):
            base = 10 * it + 3
            for k in range(10):
                g = base + k
                step(g, (3 + k) % 5, (3 + k) % 10, True, True, True)

        step(NB - 4, (NB - 4) % 5, (NB - 4) % 10, True, False, True)
        step(NB - 3, (NB - 3) % 5, (NB - 3) % 10, True, False, True)
        step(NB - 2, (NB - 2) % 5, (NB - 2) % 10, True, False, False)
        step(NB - 1, (NB - 1) % 5, (NB - 1) % 10, True, False, False)
        wait_scatter(NB - 3, (NB - 3) % 5, (NB - 3) % 10)
        wait_scatter(NB - 2, (NB - 2) % 5, (NB - 2) % 10)
        wait_scatter(NB - 1, (NB - 1) % 5, (NB - 1) % 10)

        plsc.subcore_barrier()
        pltpu.sync_copy(acc.at[pl.ds(base_row, rows_per_tile)],
                        out_hbm.at[c, pl.ds(base_row, rows_per_tile)])

    return spmm


_spmm_hid = _make_spmm(D_HID // 2)
_spmm_out = _make_spmm(D_OUT // 2)


def _dense_body(p_ref, w0_ref, b0_ref, w1_ref, b1_ref, y_ref):
    p = jnp.concatenate([p_ref[0], p_ref[1]], axis=1)
    h = jnp.maximum(
        jnp.dot(p, w0_ref[...], preferred_element_type=jnp.float32) + b0_ref[...], 0.0)
    y = jnp.dot(h, w1_ref[...], preferred_element_type=jnp.float32) + b1_ref[...]
    y_ref[0] = y[:, :D_OUT // 2]
    y_ref[1] = y[:, D_OUT // 2:]


def _dense(p, w0, b0, w1, b1):
    rows = 2048
    grid = (N_PAD // rows,)
    return pl.pallas_call(
        _dense_body,
        grid=grid,
        in_specs=[
            pl.BlockSpec((NC, rows, D_IN // 2), lambda i: (0, i, 0)),
            pl.BlockSpec((D_IN, D_HID), lambda i: (0, 0)),
            pl.BlockSpec((1, D_HID), lambda i: (0, 0)),
            pl.BlockSpec((D_HID, D_OUT), lambda i: (0, 0)),
            pl.BlockSpec((1, D_OUT), lambda i: (0, 0)),
        ],
        out_specs=pl.BlockSpec((NC, rows, D_OUT // 2), lambda i: (0, i, 0)),
        out_shape=jax.ShapeDtypeStruct((NC, N_PAD, D_OUT // 2), jnp.float32),
    )(p, w0, b0, w1, b1)


def _logsoftmax_body(q_ref, o_ref):
    z = jnp.concatenate([q_ref[0], q_ref[1]], axis=1)
    z = z - jnp.max(z, axis=1, keepdims=True)
    o_ref[...] = z - jnp.log(jnp.sum(jnp.exp(z), axis=1, keepdims=True))


def _logsoftmax(q):
    rows = 2000
    grid = (N_NODES // rows,)
    return pl.pallas_call(
        _logsoftmax_body,
        grid=grid,
        in_specs=[pl.BlockSpec((NC, rows, D_OUT // 2), lambda i: (0, i, 0))],
        out_specs=pl.BlockSpec((rows, D_OUT), lambda i: (i, 0)),
        out_shape=jax.ShapeDtypeStruct((N_NODES, D_OUT), jnp.float32),
    )(q)


def kernel(x, edge_index, edge_weight, W0, b0, W1, b1):
    src = edge_index[0].astype(jnp.int32)
    dst = edge_index[1].astype(jnp.int32)
    w_i = lax.bitcast_convert_type(edge_weight.astype(jnp.float32), jnp.int32)
    pad = E_PAD - N_EDGES
    src = jnp.concatenate([src, jnp.zeros((pad,), jnp.int32)])
    dst = jnp.concatenate([dst, jnp.zeros((pad,), jnp.int32)])
    w_i = jnp.concatenate([w_i, jnp.zeros((pad,), jnp.int32)])
    ep = (jnp.stack([src, dst, w_i], axis=0)
          .reshape(3, NS, BATCHES_PER_TILE, EDGE_BATCH)
          .transpose(1, 2, 0, 3))

    xh = x.reshape(N_NODES, NC, D_IN // 2).transpose(1, 0, 2)
    xh = jnp.concatenate(
        [xh, jnp.zeros((NC, N_PAD - N_NODES, D_IN // 2), jnp.float32)], axis=1)
    p = _spmm_hid(xh, ep)
    y = _dense(p, W0, b0.reshape(1, D_HID), W1, b1.reshape(1, D_OUT))
    q = _spmm_out(y, ep)
    return _logsoftmax(q)
